# baseline (speedup 1.0000x reference)
import functools

import jax
import jax.numpy as jnp
from jax import lax
from jax.experimental import pallas as pl
from jax.experimental.pallas import tpu as pltpu
from jax.experimental.pallas import tpu_sc as plsc

N = 800000
F = 16
D = 64
S = 50000
CLIP = 5.0

NC = 2
NS = 16
NW = NC * NS
WROWS = 25088
NVEC = WROWS // 16
CH_V = 784
CH_R = CH_V * 16
TAIL_V0 = ((NW - 1) * WROWS + 2 * CH_R - N) // 16
SP = 50176

PK = 8
FP = F * PK
DP = D * PK
BD2 = 4000
NBLK = (N // PK) // BD2


def _dense_body(x_ref, mean_ref, std_ref, w1_ref, b1_ref, g_ref, be_ref,
                w2_ref, b2_ref, wv_ref, j_ref, r_ref):
    x = x_ref[...]
    xn = jnp.clip((x - mean_ref[...]) / std_ref[...], -CLIP, CLIP)
    h = jnp.dot(xn.astype(jnp.bfloat16), w1_ref[...],
                preferred_element_type=jnp.float32) + b1_ref[...]
    h = jnp.maximum(h, 0.0)
    jm = j_ref[...]
    mu = jnp.dot(h, jm, preferred_element_type=jnp.float32)
    q = jnp.dot(h * h, jm, preferred_element_type=jnp.float32)
    var = jnp.maximum(q - mu * mu, 0.0)
    h = (h - mu) * lax.rsqrt(var + 1e-5) * g_ref[...] + be_ref[...]
    h = jnp.dot(h.astype(jnp.bfloat16), w2_ref[...],
                preferred_element_type=jnp.float32) + b2_ref[...]
    h = jnp.maximum(h, 0.0)
    r_ref[...] = jnp.dot(h, wv_ref[...], preferred_element_type=jnp.float32)


def _dense(x, norm_mean, norm_std, W1, b1, ln_g, ln_b, W2, b2, Wv):
    eye = jnp.eye(PK, dtype=jnp.float32)
    w1p = jnp.kron(eye, W1).astype(jnp.bfloat16)
    w2p = jnp.kron(eye, W2).astype(jnp.bfloat16)
    wvp = jnp.kron(eye, Wv)
    jmp = jnp.kron(eye, jnp.full((D, D), 1.0 / D, jnp.float32))
    return pl.pallas_call(
        _dense_body,
        grid=(NBLK,),
        in_specs=[
            pl.BlockSpec((BD2, FP), lambda i: (i, 0)),
            pl.BlockSpec((1, FP), lambda i: (0, 0)),
            pl.BlockSpec((1, FP), lambda i: (0, 0)),
            pl.BlockSpec((FP, DP), lambda i: (0, 0)),
            pl.BlockSpec((1, DP), lambda i: (0, 0)),
            pl.BlockSpec((1, DP), lambda i: (0, 0)),
            pl.BlockSpec((1, DP), lambda i: (0, 0)),
            pl.BlockSpec((DP, DP), lambda i: (0, 0)),
            pl.BlockSpec((1, DP), lambda i: (0, 0)),
            pl.BlockSpec((DP, PK), lambda i: (0, 0)),
            pl.BlockSpec((DP, DP), lambda i: (0, 0)),
        ],
        out_specs=pl.BlockSpec((BD2, PK), lambda i: (i, 0)),
        out_shape=jax.ShapeDtypeStruct((N // PK, PK), jnp.float32),
    )(x.reshape(N // PK, FP), jnp.tile(norm_mean, PK).reshape(1, FP),
      jnp.tile(norm_std, PK).reshape(1, FP), w1p,
      jnp.tile(b1, PK).reshape(1, DP), jnp.tile(ln_g, PK).reshape(1, DP),
      jnp.tile(ln_b, PK).reshape(1, DP), w2p,
      jnp.tile(b2, PK).reshape(1, DP), wvp, jmp)


def _scatter_body(r_hbm, ids_hbm, sums_hbm, cnt_hbm, acc_s, acc_c, rbuf, ibuf):
    wid = lax.axis_index("s") * NC + lax.axis_index("c")
    base = wid * WROWS

    zf = jnp.zeros((16,), jnp.float32)

    def zero_body(i, carry):
        for u in range(4):
            acc_s[pl.ds((i * 4 + u) * 16, 16)] = zf
            acc_c[pl.ds((i * 4 + u) * 16, 16)] = zf
        return carry

    lax.fori_loop(0, SP // 64, zero_body, 0)

    ones = jnp.ones((16,), jnp.float32)
    is_tail = wid == NW - 1

    def chunk_body(ci, carry):
        off = base + ci * CH_R
        off = jnp.where(is_tail & (ci == 1), N - CH_R, off)
        v0 = jnp.where(is_tail & (ci == 1), TAIL_V0 // 4, 0)
        pltpu.sync_copy(r_hbm.at[pl.ds(off, CH_R)], rbuf)
        pltpu.sync_copy(ids_hbm.at[pl.ds(off, CH_R)], ibuf)

        def vec_body(v, c2):
            for u in range(4):
                ids = ibuf[pl.ds((v * 4 + u) * 16, 16)]
                rv = rbuf[pl.ds((v * 4 + u) * 16, 16)]
                plsc.addupdate_scatter(acc_s, [ids], rv)
                plsc.addupdate_scatter(acc_c, [ids], ones)
            return c2

        lax.fori_loop(v0, CH_V // 4, vec_body, 0)
        return carry

    lax.fori_loop(0, NVEC // CH_V, chunk_body, 0)

    pltpu.sync_copy(acc_s, sums_hbm.at[wid])
    pltpu.sync_copy(acc_c, cnt_hbm.at[wid])


def _scatter(r_flat, ids_pad):
    mesh = plsc.VectorSubcoreMesh(core_axis_name="c", subcore_axis_name="s")
    k = functools.partial(
        pl.kernel,
        mesh=mesh,
        out_type=[
            jax.ShapeDtypeStruct((NW, SP), jnp.float32),
            jax.ShapeDtypeStruct((NW, SP), jnp.float32),
        ],
        scratch_types=[
            pltpu.VMEM((SP,), jnp.float32),
            pltpu.VMEM((SP,), jnp.float32),
            pltpu.VMEM((CH_R,), jnp.float32),
            pltpu.VMEM((CH_R,), jnp.int32),
        ],
        compiler_params=pltpu.CompilerParams(needs_layout_passes=False),
    )(_scatter_body)
    return k(r_flat, ids_pad)


CS = 1664


def _fin_body(sums_hbm, cnt_hbm, out_hbm, sbuf, cbuf, obuf):
    wid = lax.axis_index("s") * NC + lax.axis_index("c")
    c0 = jnp.minimum(wid * CS, SP - CS)
    pltpu.sync_copy(sums_hbm.at[:, pl.ds(c0, CS)], sbuf)
    pltpu.sync_copy(cnt_hbm.at[:, pl.ds(c0, CS)], cbuf)

    one = jnp.ones((16,), jnp.float32)

    def vec_body(v, carry):
        s = sbuf[0, pl.ds(v * 16, 16)]
        c = cbuf[0, pl.ds(v * 16, 16)]
        for w2 in range(1, NW):
            s = s + sbuf[w2, pl.ds(v * 16, 16)]
            c = c + cbuf[w2, pl.ds(v * 16, 16)]
        obuf[pl.ds(v * 16, 16)] = s / jnp.maximum(c, one)
        return carry

    lax.fori_loop(0, CS // 16, vec_body, 0)
    pltpu.sync_copy(obuf, out_hbm.at[pl.ds(c0, CS)])


def _finalize(sums_p, cnt_p):
    mesh = plsc.VectorSubcoreMesh(core_axis_name="c", subcore_axis_name="s")
    k = functools.partial(
        pl.kernel,
        mesh=mesh,
        out_type=jax.ShapeDtypeStruct((SP,), jnp.float32),
        scratch_types=[
            pltpu.VMEM((NW, CS), jnp.float32),
            pltpu.VMEM((NW, CS), jnp.float32),
            pltpu.VMEM((CS,), jnp.float32),
        ],
        compiler_params=pltpu.CompilerParams(needs_layout_passes=False),
    )(_fin_body)
    return k(sums_p, cnt_p)


def kernel(x, segment_ids, norm_mean, norm_std, W1, b1, ln_g, ln_b, W2, b2, Wv, bv):
    r = _dense(x, norm_mean, norm_std, W1, b1, ln_g, ln_b, W2, b2, Wv)
    seg = segment_ids.astype(jnp.int32)
    sums_p, cnt_p = _scatter(r.reshape(N), seg)
    vals = _finalize(sums_p, cnt_p)
    return (vals[:S] + bv[0]).reshape(S, 1)


# speedup vs baseline: 1.0726x; 1.0726x over previous
import functools

import jax
import jax.numpy as jnp
from jax import lax
from jax.experimental import pallas as pl
from jax.experimental.pallas import tpu as pltpu
from jax.experimental.pallas import tpu_sc as plsc

N = 800000
F = 16
D = 64
S = 50000
CLIP = 5.0

NC = 2
NS = 16
NW = NC * NS
WROWS = 25088
NVEC = WROWS // 16
CH_V = 784
CH_R = CH_V * 16
TAIL_V0 = ((NW - 1) * WROWS + 2 * CH_R - N) // 16
SP = 50176

BD = 8192
NBLK = -(-N // BD)


def _dense_body(x_ref, mean_ref, std_ref, w1_ref, b1_ref, g_ref, be_ref,
                w2_ref, b2_ref, wv_ref, j_ref, r_ref):
    x = x_ref[...]
    xn = jnp.clip((x - mean_ref[...]) / std_ref[...], -CLIP, CLIP)
    h = jnp.dot(xn.astype(jnp.bfloat16), w1_ref[...],
                preferred_element_type=jnp.float32) + b1_ref[...]
    h = jnp.maximum(h, 0.0)
    jm = j_ref[...]
    mu = jnp.dot(h, jm, preferred_element_type=jnp.float32)
    q = jnp.dot(h * h, jm, preferred_element_type=jnp.float32)
    var = jnp.maximum(q - mu * mu, 0.0)
    h = (h - mu) * lax.rsqrt(var + 1e-5) * g_ref[...] + be_ref[...]
    h = jnp.dot(h.astype(jnp.bfloat16), w2_ref[...],
                preferred_element_type=jnp.float32) + b2_ref[...]
    h = jnp.maximum(h, 0.0)
    r_row = lax.dot_general(wv_ref[...], h, (((1,), (1,)), ((), ())),
                            preferred_element_type=jnp.float32)
    r_ref[...] = r_row[None]


def _dense(x, norm_mean, norm_std, W1, b1, ln_g, ln_b, W2, b2, Wv):
    jm = jnp.full((D, D), 1.0 / D, dtype=jnp.float32)
    return pl.pallas_call(
        _dense_body,
        grid=(NBLK,),
        in_specs=[
            pl.BlockSpec((BD, F), lambda i: (i, 0)),
            pl.BlockSpec((1, F), lambda i: (0, 0)),
            pl.BlockSpec((1, F), lambda i: (0, 0)),
            pl.BlockSpec((F, D), lambda i: (0, 0)),
            pl.BlockSpec((1, D), lambda i: (0, 0)),
            pl.BlockSpec((1, D), lambda i: (0, 0)),
            pl.BlockSpec((1, D), lambda i: (0, 0)),
            pl.BlockSpec((D, D), lambda i: (0, 0)),
            pl.BlockSpec((1, D), lambda i: (0, 0)),
            pl.BlockSpec((1, D), lambda i: (0, 0)),
            pl.BlockSpec((D, D), lambda i: (0, 0)),
        ],
        out_specs=pl.BlockSpec((1, 1, BD), lambda i: (i, 0, 0)),
        out_shape=jax.ShapeDtypeStruct((NBLK, 1, BD), jnp.float32),
    )(x, norm_mean.reshape(1, F), norm_std.reshape(1, F),
      W1.astype(jnp.bfloat16), b1.reshape(1, D), ln_g.reshape(1, D),
      ln_b.reshape(1, D), W2.astype(jnp.bfloat16), b2.reshape(1, D),
      Wv.reshape(1, D), jm)


def _scatter_body(r_hbm, ids_hbm, sums_hbm, cnt_hbm, acc_s, acc_c, rbuf, ibuf):
    wid = lax.axis_index("s") * NC + lax.axis_index("c")
    base = wid * WROWS

    zf = jnp.zeros((16,), jnp.float32)

    def zero_body(i, carry):
        for u in range(4):
            acc_s[pl.ds((i * 4 + u) * 16, 16)] = zf
            acc_c[pl.ds((i * 4 + u) * 16, 16)] = zf
        return carry

    lax.fori_loop(0, SP // 64, zero_body, 0)

    ones = jnp.ones((16,), jnp.float32)
    is_tail = wid == NW - 1

    def chunk_body(ci, carry):
        off = base + ci * CH_R
        off = jnp.where(is_tail & (ci == 1), N - CH_R, off)
        v0 = jnp.where(is_tail & (ci == 1), TAIL_V0 // 4, 0)
        pltpu.sync_copy(r_hbm.at[pl.ds(off, CH_R)], rbuf)
        pltpu.sync_copy(ids_hbm.at[pl.ds(off, CH_R)], ibuf)

        def vec_body(v, c2):
            for u in range(4):
                ids = ibuf[pl.ds((v * 4 + u) * 16, 16)]
                rv = rbuf[pl.ds((v * 4 + u) * 16, 16)]
                plsc.addupdate_scatter(acc_s, [ids], rv)
                plsc.addupdate_scatter(acc_c, [ids], ones)
            return c2

        lax.fori_loop(v0, CH_V // 4, vec_body, 0)
        return carry

    lax.fori_loop(0, NVEC // CH_V, chunk_body, 0)

    pltpu.sync_copy(acc_s, sums_hbm.at[wid])
    pltpu.sync_copy(acc_c, cnt_hbm.at[wid])


def _scatter(r_flat, ids_pad):
    mesh = plsc.VectorSubcoreMesh(core_axis_name="c", subcore_axis_name="s")
    k = functools.partial(
        pl.kernel,
        mesh=mesh,
        out_type=[
            jax.ShapeDtypeStruct((NW, SP), jnp.float32),
            jax.ShapeDtypeStruct((NW, SP), jnp.float32),
        ],
        scratch_types=[
            pltpu.VMEM((SP,), jnp.float32),
            pltpu.VMEM((SP,), jnp.float32),
            pltpu.VMEM((CH_R,), jnp.float32),
            pltpu.VMEM((CH_R,), jnp.int32),
        ],
        compiler_params=pltpu.CompilerParams(needs_layout_passes=False),
    )(_scatter_body)
    return k(r_flat, ids_pad)


CS = 1664


def _fin_body(sums_hbm, cnt_hbm, out_hbm, sbuf, cbuf, obuf):
    wid = lax.axis_index("s") * NC + lax.axis_index("c")
    c0 = jnp.minimum(wid * CS, SP - CS)
    pltpu.sync_copy(sums_hbm.at[:, pl.ds(c0, CS)], sbuf)
    pltpu.sync_copy(cnt_hbm.at[:, pl.ds(c0, CS)], cbuf)

    one = jnp.ones((16,), jnp.float32)

    def vec_body(v, carry):
        s = sbuf[0, pl.ds(v * 16, 16)]
        c = cbuf[0, pl.ds(v * 16, 16)]
        for w2 in range(1, NW):
            s = s + sbuf[w2, pl.ds(v * 16, 16)]
            c = c + cbuf[w2, pl.ds(v * 16, 16)]
        obuf[pl.ds(v * 16, 16)] = s / jnp.maximum(c, one)
        return carry

    lax.fori_loop(0, CS // 16, vec_body, 0)
    pltpu.sync_copy(obuf, out_hbm.at[pl.ds(c0, CS)])


def _finalize(sums_p, cnt_p):
    mesh = plsc.VectorSubcoreMesh(core_axis_name="c", subcore_axis_name="s")
    k = functools.partial(
        pl.kernel,
        mesh=mesh,
        out_type=jax.ShapeDtypeStruct((SP,), jnp.float32),
        scratch_types=[
            pltpu.VMEM((NW, CS), jnp.float32),
            pltpu.VMEM((NW, CS), jnp.float32),
            pltpu.VMEM((CS,), jnp.float32),
        ],
        compiler_params=pltpu.CompilerParams(needs_layout_passes=False),
    )(_fin_body)
    return k(sums_p, cnt_p)


def kernel(x, segment_ids, norm_mean, norm_std, W1, b1, ln_g, ln_b, W2, b2, Wv, bv):
    r = _dense(x, norm_mean, norm_std, W1, b1, ln_g, ln_b, W2, b2, Wv)
    seg = segment_ids.astype(jnp.int32)
    sums_p, cnt_p = _scatter(r.reshape(NBLK * BD)[:N], seg)
    vals = _finalize(sums_p, cnt_p)
    return (vals[:S] + bv[0]).reshape(S, 1)


# speedup vs baseline: 1.0939x; 1.0199x over previous
import functools

import jax
import jax.numpy as jnp
from jax import lax
from jax.experimental import pallas as pl
from jax.experimental.pallas import tpu as pltpu
from jax.experimental.pallas import tpu_sc as plsc

N = 800000
F = 16
D = 64
S = 50000
CLIP = 5.0

NC = 2
NS = 16
NW = NC * NS
WROWS = 25088
NVEC = WROWS // 16
CH_V = 784
CH_R = CH_V * 16
TAIL_V0 = ((NW - 1) * WROWS + 2 * CH_R - N) // 16
SP = 50176

BD = 16384
NBLK = -(-N // BD)


def _dense_body(x_ref, mean_ref, std_ref, w1_ref, b1_ref, g_ref, be_ref,
                w2_ref, b2_ref, wv_ref, j_ref, r_ref):
    x = x_ref[...]
    xn = jnp.clip((x - mean_ref[...]) / std_ref[...], -CLIP, CLIP)
    h = jnp.dot(xn.astype(jnp.bfloat16), w1_ref[...],
                preferred_element_type=jnp.float32) + b1_ref[...]
    h = jnp.maximum(h, 0.0)
    jm = j_ref[...]
    mu = jnp.dot(h, jm, preferred_element_type=jnp.float32)
    q = jnp.dot(h * h, jm, preferred_element_type=jnp.float32)
    var = jnp.maximum(q - mu * mu, 0.0)
    h = (h - mu) * lax.rsqrt(var + 1e-5) * g_ref[...] + be_ref[...]
    h = jnp.dot(h.astype(jnp.bfloat16), w2_ref[...],
                preferred_element_type=jnp.float32) + b2_ref[...]
    h = jnp.maximum(h, 0.0)
    r_row = lax.dot_general(wv_ref[...], h, (((1,), (1,)), ((), ())),
                            preferred_element_type=jnp.float32)
    r_ref[...] = r_row[None]


def _dense(x, norm_mean, norm_std, W1, b1, ln_g, ln_b, W2, b2, Wv):
    jm = jnp.full((D, D), 1.0 / D, dtype=jnp.float32)
    return pl.pallas_call(
        _dense_body,
        grid=(NBLK,),
        in_specs=[
            pl.BlockSpec((BD, F), lambda i: (i, 0)),
            pl.BlockSpec((1, F), lambda i: (0, 0)),
            pl.BlockSpec((1, F), lambda i: (0, 0)),
            pl.BlockSpec((F, D), lambda i: (0, 0)),
            pl.BlockSpec((1, D), lambda i: (0, 0)),
            pl.BlockSpec((1, D), lambda i: (0, 0)),
            pl.BlockSpec((1, D), lambda i: (0, 0)),
            pl.BlockSpec((D, D), lambda i: (0, 0)),
            pl.BlockSpec((1, D), lambda i: (0, 0)),
            pl.BlockSpec((1, D), lambda i: (0, 0)),
            pl.BlockSpec((D, D), lambda i: (0, 0)),
        ],
        out_specs=pl.BlockSpec((1, 1, BD), lambda i: (i, 0, 0)),
        out_shape=jax.ShapeDtypeStruct((NBLK, 1, BD), jnp.float32),
    )(x, norm_mean.reshape(1, F), norm_std.reshape(1, F),
      W1.astype(jnp.bfloat16), b1.reshape(1, D), ln_g.reshape(1, D),
      ln_b.reshape(1, D), W2.astype(jnp.bfloat16), b2.reshape(1, D),
      Wv.reshape(1, D), jm)


def _scatter_body(r_hbm, ids_hbm, sums_hbm, cnt_hbm, acc_s, acc_c, rbuf, ibuf):
    wid = lax.axis_index("s") * NC + lax.axis_index("c")
    base = wid * WROWS

    zf = jnp.zeros((16,), jnp.float32)

    def zero_body(i, carry):
        for u in range(4):
            acc_s[pl.ds((i * 4 + u) * 16, 16)] = zf
            acc_c[pl.ds((i * 4 + u) * 16, 16)] = zf
        return carry

    lax.fori_loop(0, SP // 64, zero_body, 0)

    ones = jnp.ones((16,), jnp.float32)
    is_tail = wid == NW - 1

    def chunk_body(ci, carry):
        off = base + ci * CH_R
        off = jnp.where(is_tail & (ci == 1), N - CH_R, off)
        v0 = jnp.where(is_tail & (ci == 1), TAIL_V0 // 4, 0)
        pltpu.sync_copy(r_hbm.at[pl.ds(off, CH_R)], rbuf)
        pltpu.sync_copy(ids_hbm.at[pl.ds(off, CH_R)], ibuf)

        def vec_body(v, c2):
            for u in range(4):
                ids = ibuf[pl.ds((v * 4 + u) * 16, 16)]
                rv = rbuf[pl.ds((v * 4 + u) * 16, 16)]
                plsc.addupdate_scatter(acc_s, [ids], rv)
                plsc.addupdate_scatter(acc_c, [ids], ones)
            return c2

        lax.fori_loop(v0, CH_V // 4, vec_body, 0)
        return carry

    lax.fori_loop(0, NVEC // CH_V, chunk_body, 0)

    pltpu.sync_copy(acc_s, sums_hbm.at[wid])
    pltpu.sync_copy(acc_c, cnt_hbm.at[wid])


def _scatter(r_flat, ids_pad):
    mesh = plsc.VectorSubcoreMesh(core_axis_name="c", subcore_axis_name="s")
    k = functools.partial(
        pl.kernel,
        mesh=mesh,
        out_type=[
            jax.ShapeDtypeStruct((NW, SP), jnp.float32),
            jax.ShapeDtypeStruct((NW, SP), jnp.float32),
        ],
        scratch_types=[
            pltpu.VMEM((SP,), jnp.float32),
            pltpu.VMEM((SP,), jnp.float32),
            pltpu.VMEM((CH_R,), jnp.float32),
            pltpu.VMEM((CH_R,), jnp.int32),
        ],
        compiler_params=pltpu.CompilerParams(needs_layout_passes=False),
    )(_scatter_body)
    return k(r_flat, ids_pad)


CS = 1664


def _fin_body(sums_hbm, cnt_hbm, out_hbm, sbuf, cbuf, obuf):
    wid = lax.axis_index("s") * NC + lax.axis_index("c")
    c0 = jnp.minimum(wid * CS, SP - CS)
    pltpu.sync_copy(sums_hbm.at[:, pl.ds(c0, CS)], sbuf)
    pltpu.sync_copy(cnt_hbm.at[:, pl.ds(c0, CS)], cbuf)

    one = jnp.ones((16,), jnp.float32)

    def vec_body(v, carry):
        s = sbuf[0, pl.ds(v * 16, 16)]
        c = cbuf[0, pl.ds(v * 16, 16)]
        for w2 in range(1, NW):
            s = s + sbuf[w2, pl.ds(v * 16, 16)]
            c = c + cbuf[w2, pl.ds(v * 16, 16)]
        obuf[pl.ds(v * 16, 16)] = s / jnp.maximum(c, one)
        return carry

    lax.fori_loop(0, CS // 16, vec_body, 0)
    pltpu.sync_copy(obuf, out_hbm.at[pl.ds(c0, CS)])


def _finalize(sums_p, cnt_p):
    mesh = plsc.VectorSubcoreMesh(core_axis_name="c", subcore_axis_name="s")
    k = functools.partial(
        pl.kernel,
        mesh=mesh,
        out_type=jax.ShapeDtypeStruct((SP,), jnp.float32),
        scratch_types=[
            pltpu.VMEM((NW, CS), jnp.float32),
            pltpu.VMEM((NW, CS), jnp.float32),
            pltpu.VMEM((CS,), jnp.float32),
        ],
        compiler_params=pltpu.CompilerParams(needs_layout_passes=False),
    )(_fin_body)
    return k(sums_p, cnt_p)


def kernel(x, segment_ids, norm_mean, norm_std, W1, b1, ln_g, ln_b, W2, b2, Wv, bv):
    r = _dense(x, norm_mean, norm_std, W1, b1, ln_g, ln_b, W2, b2, Wv)
    seg = segment_ids.astype(jnp.int32)
    sums_p, cnt_p = _scatter(r.reshape(NBLK * BD)[:N], seg)
    vals = _finalize(sums_p, cnt_p)
    return (vals[:S] + bv[0]).reshape(S, 1)
